# Initial kernel scaffold; baseline (speedup 1.0000x reference)
import jax
import jax.numpy as jnp
from jax.experimental import pallas as pl


def kernel(x, edge_index, w0, b0, c1_w1, c1_b1, c1_g, c1_be, c1_w2, c1_b2, c2_w1, c2_b1, c2_g, c2_be, c2_w2, c2_b2, c3_w1, c3_b1, c3_g, c3_be, c3_w2, c3_b2, w4, b4):
    raise NotImplementedError("write your pallas kernel here")


# speedup vs baseline: 13.7940x; 13.7940x over previous
import functools

import jax
import jax.numpy as jnp
from jax import lax
from jax.experimental import pallas as pl
from jax.experimental.pallas import tpu as pltpu
from jax.experimental.pallas import tpu_sc as plsc

N = 10000
E = 320000
D_IN = 128
D_OUT = 128
H = 32

NC, NS = 2, 16
NW = NC * NS
CH = 128
NCHUNK = 80
EPW = NCHUNK * CH
EP = NW * EPW
NP = 10112
RPS = NP // NS

@functools.cache
def _get_sc_aggregate():
    mesh = plsc.VectorSubcoreMesh(
        core_axis_name="c", subcore_axis_name="s",
        num_cores=NC, num_subcores=NS)

    @functools.partial(
        pl.kernel,
        out_type=jax.ShapeDtypeStruct((NC, NP, 2 * H), jnp.float32),
        mesh=mesh,
        scratch_types=[
            pltpu.VMEM((NCHUNK, CH), jnp.int32),
            pltpu.VMEM((NCHUNK, CH), jnp.int32),
            pltpu.VMEM((CH, 2 * H), jnp.float32),
            pltpu.VMEM_SHARED((NP, 2 * H), jnp.float32),
            pltpu.SemaphoreType.DMA,
        ],
        compiler_params=pltpu.CompilerParams(use_tc_tiling_on_sc=False),
    )
    def _sc_aggregate(u_hbm, src_hbm, dst_hbm, z_hbm, out_hbm,
                      src_v, dst_v, rows_v, acc_sh, sem):
        c = lax.axis_index("c")
        s = lax.axis_index("s")
        wid = c * NS + s
        pltpu.sync_copy(z_hbm.at[pl.ds(s * RPS, RPS)],
                        acc_sh.at[pl.ds(s * RPS, RPS)])
        pltpu.sync_copy(src_hbm.at[wid], src_v)
        pltpu.sync_copy(dst_hbm.at[wid], dst_v)
        plsc.subcore_barrier()

        def chunk_body(j, carry):
            pltpu.async_copy(u_hbm.at[src_v.at[j]], rows_v, sem).wait()
            pltpu.sync_copy(rows_v, acc_sh.at[dst_v.at[j]], add=True)
            return carry

        lax.fori_loop(0, NCHUNK, chunk_body, 0)
        plsc.subcore_barrier()
        pltpu.sync_copy(acc_sh.at[pl.ds(s * RPS, RPS)],
                        out_hbm.at[c, pl.ds(s * RPS, RPS)])

    return _sc_aggregate


def _build_u(xn, u_ref):
    r = jnp.maximum(xn, 0.0) + 1e-7
    g = jnp.max(r)
    q = jnp.exp(r - g)
    u_ref[0:N, :] = jnp.concatenate([q, q * r], axis=1)
    u_ref[N:NP, :] = jnp.zeros((NP - N, 2 * H), jnp.float32)


def _pre_body(x_ref, w0_ref, b0_ref, x0_ref, u_ref):
    x0 = jnp.dot(x_ref[...], w0_ref[...],
                 preferred_element_type=jnp.float32) + b0_ref[...]
    x0_ref[...] = x0
    _build_u(x0, u_ref)


_pre_call = pl.pallas_call(
    _pre_body,
    out_shape=(jax.ShapeDtypeStruct((N, H), jnp.float32),
               jax.ShapeDtypeStruct((NP, 2 * H), jnp.float32)),
)


def _conv_dense(part, x_in, w1, b1, g, be, w2, b2):
    s = part[0, 0:N, :] + part[1, 0:N, :]
    s0 = s[:, 0:H]
    s1 = s[:, H:2 * H]
    aggr = s1 / (s0 + 1e-30)
    h = aggr + x_in
    h1 = jnp.dot(h, w1, preferred_element_type=jnp.float32) + b1
    mean = jnp.mean(h1, axis=0, keepdims=True)
    d = h1 - mean
    var = jnp.mean(d * d, axis=0, keepdims=True)
    hn = d * (g / jnp.sqrt(var + 1e-5)) + be
    hr = jnp.maximum(hn, 0.0)
    h2 = jnp.dot(hr, w2, preferred_element_type=jnp.float32) + b2
    return jnp.maximum(h2, 0.0)


def _mid_body(part_ref, x_ref, w1_ref, b1_ref, g_ref, be_ref, w2_ref, b2_ref,
              xn_ref, u_ref):
    xn = _conv_dense(part_ref[...], x_ref[...], w1_ref[...], b1_ref[...],
                     g_ref[...], be_ref[...], w2_ref[...], b2_ref[...])
    xn_ref[...] = xn
    _build_u(xn, u_ref)


_mid_call = pl.pallas_call(
    _mid_body,
    out_shape=(jax.ShapeDtypeStruct((N, H), jnp.float32),
               jax.ShapeDtypeStruct((NP, 2 * H), jnp.float32)),
)


def _last_body(part_ref, x_ref, w1_ref, b1_ref, g_ref, be_ref, w2_ref, b2_ref,
               w4_ref, b4_ref, out_ref):
    x3 = _conv_dense(part_ref[...], x_ref[...], w1_ref[...], b1_ref[...],
                     g_ref[...], be_ref[...], w2_ref[...], b2_ref[...])
    out_ref[...] = jnp.dot(x3, w4_ref[...],
                           preferred_element_type=jnp.float32) + b4_ref[...]


_last_call = pl.pallas_call(
    _last_body,
    out_shape=jax.ShapeDtypeStruct((N, D_OUT), jnp.float32),
)


def kernel(x, edge_index, w0, b0,
           c1_w1, c1_b1, c1_g, c1_be, c1_w2, c1_b2,
           c2_w1, c2_b1, c2_g, c2_be, c2_w2, c2_b2,
           c3_w1, c3_b1, c3_g, c3_be, c3_w2, c3_b2,
           w4, b4):
    pad = jnp.full((EP - E,), N, jnp.int32)
    srcr = jnp.concatenate([edge_index[0], pad]).reshape(NW, NCHUNK, CH)
    dstr = jnp.concatenate([edge_index[1], pad]).reshape(NW, NCHUNK, CH)
    z = jnp.zeros((NP, 2 * H), jnp.float32)

    x0, u = _pre_call(x, w0, b0.reshape(1, H))
    convs = [
        (c1_w1, c1_b1, c1_g, c1_be, c1_w2, c1_b2),
        (c2_w1, c2_b1, c2_g, c2_be, c2_w2, c2_b2),
        (c3_w1, c3_b1, c3_g, c3_be, c3_w2, c3_b2),
    ]
    sc_aggregate = _get_sc_aggregate()
    xc = x0
    for layer, (w1, b1, g, be, w2, b2) in enumerate(convs):
        part = sc_aggregate(u, srcr, dstr, z)
        args = (part, xc, w1, b1.reshape(1, 2 * H), g.reshape(1, 2 * H),
                be.reshape(1, 2 * H), w2, b2.reshape(1, H))
        if layer < 2:
            xc, u = _mid_call(*args)
        else:
            out = _last_call(*args, w4, b4.reshape(1, D_OUT))
    return out


# speedup vs baseline: 15.5275x; 1.1257x over previous
import functools

import jax
import jax.numpy as jnp
from jax import lax
from jax.experimental import pallas as pl
from jax.experimental.pallas import tpu as pltpu
from jax.experimental.pallas import tpu_sc as plsc

N = 10000
E = 320000
D_IN = 128
D_OUT = 128
H = 32

NC, NS = 2, 16
NW = NC * NS
CH = 128
NCHUNK = 80
EPW = NCHUNK * CH
EP = NW * EPW
NP = 10112
RPS = NP // NS

@functools.cache
def _get_sc_aggregate():
    mesh = plsc.VectorSubcoreMesh(
        core_axis_name="c", subcore_axis_name="s",
        num_cores=NC, num_subcores=NS)

    @functools.partial(
        pl.kernel,
        out_type=jax.ShapeDtypeStruct((NC, NP, 2 * H), jnp.float32),
        mesh=mesh,
        scratch_types=[
            pltpu.VMEM((NCHUNK, CH), jnp.int32),
            pltpu.VMEM((NCHUNK, CH), jnp.int32),
            pltpu.VMEM((CH, 2 * H), jnp.float32),
            pltpu.VMEM((CH, 2 * H), jnp.float32),
            pltpu.VMEM_SHARED((NP, 2 * H), jnp.float32),
            pltpu.SemaphoreType.DMA,
            pltpu.SemaphoreType.DMA,
            pltpu.SemaphoreType.DMA,
            pltpu.SemaphoreType.DMA,
        ],
        compiler_params=pltpu.CompilerParams(use_tc_tiling_on_sc=False),
    )
    def _sc_aggregate(u_hbm, src_hbm, dst_hbm, z_hbm, out_hbm,
                      src_v, dst_v, rows_a, rows_b, acc_sh, ga, gb, sa, sb):
        c = lax.axis_index("c")
        s = lax.axis_index("s")
        wid = c * NS + s
        pltpu.sync_copy(z_hbm.at[pl.ds(s * RPS, RPS)],
                        acc_sh.at[pl.ds(s * RPS, RPS)])
        pltpu.sync_copy(src_hbm.at[wid], src_v)
        pltpu.sync_copy(dst_hbm.at[wid], dst_v)
        plsc.subcore_barrier()

        def g_start(j, buf, sem):
            pltpu.async_copy(u_hbm.at[src_v.at[j]], buf, sem)

        def g_wait(j, buf, sem):
            pltpu.make_async_copy(u_hbm.at[src_v.at[j]], buf, sem).wait()

        def s_start(j, buf, sem):
            pltpu.async_copy(buf, acc_sh.at[dst_v.at[j]], sem, add=True)

        def s_wait(j, buf, sem):
            pltpu.make_async_copy(buf, acc_sh.at[dst_v.at[j]], sem).wait()

        g_start(0, rows_a, ga)
        g_start(1, rows_b, gb)
        g_wait(0, rows_a, ga)
        s_start(0, rows_a, sa)
        g_wait(1, rows_b, gb)
        s_start(1, rows_b, sb)

        def pair_body(i, carry):
            j0 = 2 * i
            j1 = j0 + 1
            s_wait(j0, rows_a, sa)
            g_start(j0, rows_a, ga)
            s_wait(j1, rows_b, sb)
            g_start(j1, rows_b, gb)
            g_wait(j0, rows_a, ga)
            s_start(j0, rows_a, sa)
            g_wait(j1, rows_b, gb)
            s_start(j1, rows_b, sb)
            return carry

        lax.fori_loop(1, NCHUNK // 2, pair_body, 0)
        s_wait(NCHUNK - 2, rows_a, sa)
        s_wait(NCHUNK - 1, rows_b, sb)
        plsc.subcore_barrier()
        pltpu.sync_copy(acc_sh.at[pl.ds(s * RPS, RPS)],
                        out_hbm.at[c, pl.ds(s * RPS, RPS)])

    return _sc_aggregate


def _build_u(xn, u_ref):
    r = jnp.maximum(xn, 0.0) + 1e-7
    g = jnp.max(r)
    q = jnp.exp(r - g)
    u_ref[0:N, :] = jnp.concatenate([q, q * r], axis=1)
    u_ref[N:NP, :] = jnp.zeros((NP - N, 2 * H), jnp.float32)


def _pre_body(x_ref, w0_ref, b0_ref, x0_ref, u_ref):
    x0 = jnp.dot(x_ref[...], w0_ref[...],
                 preferred_element_type=jnp.float32) + b0_ref[...]
    x0_ref[...] = x0
    _build_u(x0, u_ref)


_pre_call = pl.pallas_call(
    _pre_body,
    out_shape=(jax.ShapeDtypeStruct((N, H), jnp.float32),
               jax.ShapeDtypeStruct((NP, 2 * H), jnp.float32)),
)


def _conv_dense(part, x_in, w1, b1, g, be, w2, b2):
    s = part[0, 0:N, :] + part[1, 0:N, :]
    s0 = s[:, 0:H]
    s1 = s[:, H:2 * H]
    aggr = s1 / (s0 + 1e-30)
    h = aggr + x_in
    h1 = jnp.dot(h, w1, preferred_element_type=jnp.float32) + b1
    mean = jnp.mean(h1, axis=0, keepdims=True)
    d = h1 - mean
    var = jnp.mean(d * d, axis=0, keepdims=True)
    hn = d * (g / jnp.sqrt(var + 1e-5)) + be
    hr = jnp.maximum(hn, 0.0)
    h2 = jnp.dot(hr, w2, preferred_element_type=jnp.float32) + b2
    return jnp.maximum(h2, 0.0)


def _mid_body(part_ref, x_ref, w1_ref, b1_ref, g_ref, be_ref, w2_ref, b2_ref,
              xn_ref, u_ref):
    xn = _conv_dense(part_ref[...], x_ref[...], w1_ref[...], b1_ref[...],
                     g_ref[...], be_ref[...], w2_ref[...], b2_ref[...])
    xn_ref[...] = xn
    _build_u(xn, u_ref)


_mid_call = pl.pallas_call(
    _mid_body,
    out_shape=(jax.ShapeDtypeStruct((N, H), jnp.float32),
               jax.ShapeDtypeStruct((NP, 2 * H), jnp.float32)),
)


def _last_body(part_ref, x_ref, w1_ref, b1_ref, g_ref, be_ref, w2_ref, b2_ref,
               w4_ref, b4_ref, out_ref):
    x3 = _conv_dense(part_ref[...], x_ref[...], w1_ref[...], b1_ref[...],
                     g_ref[...], be_ref[...], w2_ref[...], b2_ref[...])
    out_ref[...] = jnp.dot(x3, w4_ref[...],
                           preferred_element_type=jnp.float32) + b4_ref[...]


_last_call = pl.pallas_call(
    _last_body,
    out_shape=jax.ShapeDtypeStruct((N, D_OUT), jnp.float32),
)


def kernel(x, edge_index, w0, b0,
           c1_w1, c1_b1, c1_g, c1_be, c1_w2, c1_b2,
           c2_w1, c2_b1, c2_g, c2_be, c2_w2, c2_b2,
           c3_w1, c3_b1, c3_g, c3_be, c3_w2, c3_b2,
           w4, b4):
    pad = jnp.full((EP - E,), N, jnp.int32)
    srcr = jnp.concatenate([edge_index[0], pad]).reshape(NW, NCHUNK, CH)
    dstr = jnp.concatenate([edge_index[1], pad]).reshape(NW, NCHUNK, CH)
    z = jnp.zeros((NP, 2 * H), jnp.float32)

    x0, u = _pre_call(x, w0, b0.reshape(1, H))
    convs = [
        (c1_w1, c1_b1, c1_g, c1_be, c1_w2, c1_b2),
        (c2_w1, c2_b1, c2_g, c2_be, c2_w2, c2_b2),
        (c3_w1, c3_b1, c3_g, c3_be, c3_w2, c3_b2),
    ]
    sc_aggregate = _get_sc_aggregate()
    xc = x0
    for layer, (w1, b1, g, be, w2, b2) in enumerate(convs):
        part = sc_aggregate(u, srcr, dstr, z)
        args = (part, xc, w1, b1.reshape(1, 2 * H), g.reshape(1, 2 * H),
                be.reshape(1, 2 * H), w2, b2.reshape(1, H))
        if layer < 2:
            xc, u = _mid_call(*args)
        else:
            out = _last_call(*args, w4, b4.reshape(1, D_OUT))
    return out


# speedup vs baseline: 33.5632x; 2.1615x over previous
import functools

import jax
import jax.numpy as jnp
from jax import lax
from jax.experimental import pallas as pl
from jax.experimental.pallas import tpu as pltpu
from jax.experimental.pallas import tpu_sc as plsc

N = 10000
E = 320000
D_IN = 128
D_OUT = 128
H = 32

NC, NS = 2, 16
NW = NC * NS
CH = 128
NCHUNK = 80
EPW = NCHUNK * CH
EP = NW * EPW
NP = 10112
RPS = NP // NS

@functools.cache
def _get_sc_aggregate():
    mesh = plsc.VectorSubcoreMesh(
        core_axis_name="c", subcore_axis_name="s",
        num_cores=NC, num_subcores=NS)

    @functools.partial(
        pl.kernel,
        out_type=jax.ShapeDtypeStruct((NC, NP, 2 * H), jnp.float32),
        mesh=mesh,
        scratch_types=[
            pltpu.VMEM((NCHUNK, CH), jnp.int32),
            pltpu.VMEM((NCHUNK, CH), jnp.int32),
            pltpu.VMEM((CH, 2 * H), jnp.float32),
            pltpu.VMEM((CH, 2 * H), jnp.float32),
            pltpu.VMEM_SHARED((NP, 2 * H), jnp.float32),
            pltpu.VMEM_SHARED((NP, 2 * H), jnp.float32),
            pltpu.SemaphoreType.DMA,
            pltpu.SemaphoreType.DMA,
            pltpu.SemaphoreType.DMA,
            pltpu.SemaphoreType.DMA,
        ],
        compiler_params=pltpu.CompilerParams(use_tc_tiling_on_sc=False),
    )
    def _sc_aggregate(u_hbm, src_hbm, dst_hbm, z_hbm, out_hbm,
                      src_v, dst_v, rows_a, rows_b, acc_sh, u_sh,
                      ga, gb, sa, sb):
        c = lax.axis_index("c")
        s = lax.axis_index("s")
        wid = c * NS + s
        pltpu.sync_copy(z_hbm.at[pl.ds(s * RPS, RPS)],
                        acc_sh.at[pl.ds(s * RPS, RPS)])
        pltpu.sync_copy(u_hbm.at[pl.ds(s * RPS, RPS)],
                        u_sh.at[pl.ds(s * RPS, RPS)])
        pltpu.sync_copy(src_hbm.at[wid], src_v)
        pltpu.sync_copy(dst_hbm.at[wid], dst_v)
        plsc.subcore_barrier()

        def g_start(j, buf, sem):
            pltpu.async_copy(u_sh.at[src_v.at[j]], buf, sem)

        def g_wait(j, buf, sem):
            pltpu.make_async_copy(u_sh.at[src_v.at[j]], buf, sem).wait()

        def s_start(j, buf, sem):
            pltpu.async_copy(buf, acc_sh.at[dst_v.at[j]], sem, add=True)

        def s_wait(j, buf, sem):
            pltpu.make_async_copy(buf, acc_sh.at[dst_v.at[j]], sem).wait()

        g_start(0, rows_a, ga)
        g_start(1, rows_b, gb)
        g_wait(0, rows_a, ga)
        s_start(0, rows_a, sa)
        g_wait(1, rows_b, gb)
        s_start(1, rows_b, sb)

        def pair_body(i, carry):
            j0 = 2 * i
            j1 = j0 + 1
            s_wait(j0, rows_a, sa)
            g_start(j0, rows_a, ga)
            s_wait(j1, rows_b, sb)
            g_start(j1, rows_b, gb)
            g_wait(j0, rows_a, ga)
            s_start(j0, rows_a, sa)
            g_wait(j1, rows_b, gb)
            s_start(j1, rows_b, sb)
            return carry

        lax.fori_loop(1, NCHUNK // 2, pair_body, 0)
        s_wait(NCHUNK - 2, rows_a, sa)
        s_wait(NCHUNK - 1, rows_b, sb)
        plsc.subcore_barrier()
        pltpu.sync_copy(acc_sh.at[pl.ds(s * RPS, RPS)],
                        out_hbm.at[c, pl.ds(s * RPS, RPS)])

    return _sc_aggregate


def _build_u(xn, u_ref):
    r = jnp.maximum(xn, 0.0) + 1e-7
    g = jnp.max(r)
    q = jnp.exp(r - g)
    u_ref[0:N, :] = jnp.concatenate([q, q * r], axis=1)
    u_ref[N:NP, :] = jnp.zeros((NP - N, 2 * H), jnp.float32)


def _pre_body(x_ref, w0_ref, b0_ref, x0_ref, u_ref):
    x0 = jnp.dot(x_ref[...], w0_ref[...],
                 preferred_element_type=jnp.float32) + b0_ref[...]
    x0_ref[...] = x0
    _build_u(x0, u_ref)


_pre_call = pl.pallas_call(
    _pre_body,
    out_shape=(jax.ShapeDtypeStruct((N, H), jnp.float32),
               jax.ShapeDtypeStruct((NP, 2 * H), jnp.float32)),
)


def _conv_dense(part, x_in, w1, b1, g, be, w2, b2):
    s = part[0, 0:N, :] + part[1, 0:N, :]
    s0 = s[:, 0:H]
    s1 = s[:, H:2 * H]
    aggr = s1 / (s0 + 1e-30)
    h = aggr + x_in
    h1 = jnp.dot(h, w1, preferred_element_type=jnp.float32) + b1
    mean = jnp.mean(h1, axis=0, keepdims=True)
    d = h1 - mean
    var = jnp.mean(d * d, axis=0, keepdims=True)
    hn = d * (g / jnp.sqrt(var + 1e-5)) + be
    hr = jnp.maximum(hn, 0.0)
    h2 = jnp.dot(hr, w2, preferred_element_type=jnp.float32) + b2
    return jnp.maximum(h2, 0.0)


def _mid_body(part_ref, x_ref, w1_ref, b1_ref, g_ref, be_ref, w2_ref, b2_ref,
              xn_ref, u_ref):
    xn = _conv_dense(part_ref[...], x_ref[...], w1_ref[...], b1_ref[...],
                     g_ref[...], be_ref[...], w2_ref[...], b2_ref[...])
    xn_ref[...] = xn
    _build_u(xn, u_ref)


_mid_call = pl.pallas_call(
    _mid_body,
    out_shape=(jax.ShapeDtypeStruct((N, H), jnp.float32),
               jax.ShapeDtypeStruct((NP, 2 * H), jnp.float32)),
)


def _last_body(part_ref, x_ref, w1_ref, b1_ref, g_ref, be_ref, w2_ref, b2_ref,
               w4_ref, b4_ref, out_ref):
    x3 = _conv_dense(part_ref[...], x_ref[...], w1_ref[...], b1_ref[...],
                     g_ref[...], be_ref[...], w2_ref[...], b2_ref[...])
    out_ref[...] = jnp.dot(x3, w4_ref[...],
                           preferred_element_type=jnp.float32) + b4_ref[...]


_last_call = pl.pallas_call(
    _last_body,
    out_shape=jax.ShapeDtypeStruct((N, D_OUT), jnp.float32),
)


def kernel(x, edge_index, w0, b0,
           c1_w1, c1_b1, c1_g, c1_be, c1_w2, c1_b2,
           c2_w1, c2_b1, c2_g, c2_be, c2_w2, c2_b2,
           c3_w1, c3_b1, c3_g, c3_be, c3_w2, c3_b2,
           w4, b4):
    pad = jnp.full((EP - E,), N, jnp.int32)
    srcr = jnp.concatenate([edge_index[0], pad]).reshape(NW, NCHUNK, CH)
    dstr = jnp.concatenate([edge_index[1], pad]).reshape(NW, NCHUNK, CH)
    z = jnp.zeros((NP, 2 * H), jnp.float32)

    x0, u = _pre_call(x, w0, b0.reshape(1, H))
    convs = [
        (c1_w1, c1_b1, c1_g, c1_be, c1_w2, c1_b2),
        (c2_w1, c2_b1, c2_g, c2_be, c2_w2, c2_b2),
        (c3_w1, c3_b1, c3_g, c3_be, c3_w2, c3_b2),
    ]
    sc_aggregate = _get_sc_aggregate()
    xc = x0
    for layer, (w1, b1, g, be, w2, b2) in enumerate(convs):
        part = sc_aggregate(u, srcr, dstr, z)
        args = (part, xc, w1, b1.reshape(1, 2 * H), g.reshape(1, 2 * H),
                be.reshape(1, 2 * H), w2, b2.reshape(1, H))
        if layer < 2:
            xc, u = _mid_call(*args)
        else:
            out = _last_call(*args, w4, b4.reshape(1, D_OUT))
    return out
